# Initial kernel scaffold; baseline (speedup 1.0000x reference)
import jax
import jax.numpy as jnp
from jax.experimental import pallas as pl


def kernel(tokens, notes, genres, token_table, note_table, genre_table, W, b):
    raise NotImplementedError("write your pallas kernel here")


# speedup vs baseline: 1.8388x; 1.8388x over previous
import functools

import jax
import jax.numpy as jnp
from jax import lax
from jax.experimental import pallas as pl
from jax.experimental.pallas import tpu as pltpu
from jax.experimental.pallas import tpu_sc as plsc

_B = 4096
_T = 200
_N = _B * _T
_TOK_D = 64
_NOTE_D = 32
_GEN_D = 32
_ENC = 128

_NC = 2
_NS = 16
_NW = _NC * _NS
_TPW = _N // _NW
_GPW = _B // _NW
_CHUNK = 512


def _gather_body(token_table, tokens, note_table, notes, genre_table, genres,
                 tok_out, note_out, gen_out,
                 tidx, trows, nidx, nrows, gidx, grows, sem):
    wid = lax.axis_index("s") * _NC + lax.axis_index("c")
    tbase = wid * _TPW

    def tok_step(i, carry):
        off = tbase + i * _CHUNK
        pltpu.sync_copy(tokens.at[pl.ds(off, _CHUNK)], tidx)
        pltpu.async_copy(token_table.at[tidx], trows, sem).wait()
        pltpu.sync_copy(trows, tok_out.at[pl.ds(off, _CHUNK)])
        return carry

    lax.fori_loop(0, _TPW // _CHUNK, tok_step, 0)

    def note_step(i, carry):
        off = tbase + i * _CHUNK
        pltpu.sync_copy(notes.at[pl.ds(off, _CHUNK)], nidx)
        pltpu.async_copy(note_table.at[nidx], nrows, sem).wait()
        pltpu.sync_copy(nrows, note_out.at[pl.ds(off, _CHUNK)])
        return carry

    lax.fori_loop(0, _TPW // _CHUNK, note_step, 0)

    gbase = wid * _GPW
    pltpu.sync_copy(genres.at[pl.ds(gbase, _GPW)], gidx)
    pltpu.async_copy(genre_table.at[gidx], grows, sem).wait()
    pltpu.sync_copy(grows, gen_out.at[pl.ds(gbase, _GPW)])


def _sc_gather(token_table, tokens_flat, note_table, notes_flat,
               genre_table, genres):
    mesh = plsc.VectorSubcoreMesh(core_axis_name="c", subcore_axis_name="s")
    k = pl.kernel(
        _gather_body,
        mesh=mesh,
        compiler_params=pltpu.CompilerParams(use_tc_tiling_on_sc=False),
        out_type=[
            jax.ShapeDtypeStruct((_N, _TOK_D), jnp.float32),
            jax.ShapeDtypeStruct((_N, _NOTE_D), jnp.float32),
            jax.ShapeDtypeStruct((_B, _GEN_D), jnp.float32),
        ],
        scratch_types=[
            pltpu.VMEM((_CHUNK,), jnp.int32),
            pltpu.VMEM((_CHUNK, _TOK_D), jnp.float32),
            pltpu.VMEM((_CHUNK,), jnp.int32),
            pltpu.VMEM((_CHUNK, _NOTE_D), jnp.float32),
            pltpu.VMEM((_GPW,), jnp.int32),
            pltpu.VMEM((_GPW, _GEN_D), jnp.float32),
            pltpu.SemaphoreType.DMA,
        ],
    )
    return k(token_table, tokens_flat, note_table, notes_flat,
             genre_table, genres)


_RBLK = 8
_ROWS = _RBLK * _T


def _proj_body(tok_ref, note_ref, gen_ref, wt_tok_ref, wt_note_ref,
               wt_gen_ref, bias_ref, out_ref):
    acc = jnp.dot(tok_ref[...], wt_tok_ref[...],
                  preferred_element_type=jnp.float32)
    acc += jnp.dot(note_ref[...], wt_note_ref[...],
                   preferred_element_type=jnp.float32)
    gen_c = jnp.dot(gen_ref[...], wt_gen_ref[...],
                    preferred_element_type=jnp.float32) + bias_ref[...]
    row = lax.broadcasted_iota(jnp.int32, (_ROWS, _RBLK), 0) // _T
    col = lax.broadcasted_iota(jnp.int32, (_ROWS, _RBLK), 1)
    onehot = (row == col).astype(jnp.float32)
    acc += jnp.dot(onehot, gen_c, preferred_element_type=jnp.float32)
    out_ref[...] = acc


def _tc_project(tok_e, note_e, gen_rows, wt_tok, wt_note, wt_gen, bias2d):
    grid = (_N // _ROWS,)
    return pl.pallas_call(
        _proj_body,
        grid=grid,
        in_specs=[
            pl.BlockSpec((_ROWS, _TOK_D), lambda i: (i, 0)),
            pl.BlockSpec((_ROWS, _NOTE_D), lambda i: (i, 0)),
            pl.BlockSpec((_RBLK, _GEN_D), lambda i: (i, 0)),
            pl.BlockSpec((_TOK_D, _ENC), lambda i: (0, 0)),
            pl.BlockSpec((_NOTE_D, _ENC), lambda i: (0, 0)),
            pl.BlockSpec((_GEN_D, _ENC), lambda i: (0, 0)),
            pl.BlockSpec((1, _ENC), lambda i: (0, 0)),
        ],
        out_specs=pl.BlockSpec((_ROWS, _ENC), lambda i: (i, 0)),
        out_shape=jax.ShapeDtypeStruct((_N, _ENC), jnp.float32),
    )(tok_e, note_e, gen_rows, wt_tok, wt_note, wt_gen, bias2d)


def kernel(tokens, notes, genres, token_table, note_table, genre_table, W, b):
    tok_e, note_e, gen_rows = _sc_gather(
        token_table, tokens.reshape(-1), note_table, notes.reshape(-1),
        genre_table, genres)
    wt_tok = W[:, :_TOK_D].T
    wt_note = W[:, _TOK_D:_TOK_D + _NOTE_D].T
    wt_gen = W[:, _TOK_D + _NOTE_D:].T
    out = _tc_project(tok_e, note_e, gen_rows, wt_tok, wt_note, wt_gen,
                      b.reshape(1, _ENC))
    return out.reshape(_B, _T, _ENC)


# speedup vs baseline: 2.0405x; 1.1097x over previous
import jax
import jax.numpy as jnp
from jax import lax
from jax.experimental import pallas as pl
from jax.experimental.pallas import tpu as pltpu
from jax.experimental.pallas import tpu_sc as plsc

_B = 4096
_T = 200
_N = _B * _T
_TOK_D = 64
_NOTE_D = 32
_GEN_D = 32
_ENC = 128

_NC = 2
_NS = 16
_NW = _NC * _NS
_TPW = _N // _NW
_CHUNK = 512


def _gather_body(token_table, tokens, note_table, notes, genre_table, gens,
                 x_out, tidx, nidx, gidx, trows, nrows, grows, sem):
    wid = lax.axis_index("s") * _NC + lax.axis_index("c")
    base = wid * _TPW

    def step(i, carry):
        off = base + i * _CHUNK
        pltpu.sync_copy(tokens.at[pl.ds(off, _CHUNK)], tidx)
        pltpu.sync_copy(notes.at[pl.ds(off, _CHUNK)], nidx)
        pltpu.sync_copy(gens.at[pl.ds(off, _CHUNK)], gidx)
        ct = pltpu.async_copy(token_table.at[tidx], trows, sem)
        cn = pltpu.async_copy(note_table.at[nidx], nrows, sem)
        cg = pltpu.async_copy(genre_table.at[gidx], grows, sem)
        ct.wait()
        cn.wait()
        cg.wait()
        pltpu.sync_copy(trows, x_out.at[pl.ds(off, _CHUNK), pl.ds(0, _TOK_D)])
        pltpu.sync_copy(nrows, x_out.at[pl.ds(off, _CHUNK),
                                        pl.ds(_TOK_D, _NOTE_D)])
        pltpu.sync_copy(grows, x_out.at[pl.ds(off, _CHUNK),
                                        pl.ds(_TOK_D + _NOTE_D, _GEN_D)])
        return carry

    lax.fori_loop(0, _TPW // _CHUNK, step, 0)


def _sc_gather(token_table, tokens_flat, note_table, notes_flat,
               genre_table, gens_flat):
    mesh = plsc.VectorSubcoreMesh(core_axis_name="c", subcore_axis_name="s")
    k = pl.kernel(
        _gather_body,
        mesh=mesh,
        compiler_params=pltpu.CompilerParams(use_tc_tiling_on_sc=False),
        out_type=[
            jax.ShapeDtypeStruct((_N, _ENC), jnp.float32),
        ],
        scratch_types=[
            pltpu.VMEM((_CHUNK,), jnp.int32),
            pltpu.VMEM((_CHUNK,), jnp.int32),
            pltpu.VMEM((_CHUNK,), jnp.int32),
            pltpu.VMEM((_CHUNK, _TOK_D), jnp.float32),
            pltpu.VMEM((_CHUNK, _NOTE_D), jnp.float32),
            pltpu.VMEM((_CHUNK, _GEN_D), jnp.float32),
            pltpu.SemaphoreType.DMA,
        ],
    )
    return k(token_table, tokens_flat, note_table, notes_flat,
             genre_table, gens_flat)


_ROWS = 1600


def _proj_body(x_ref, wt_ref, bias_ref, out_ref):
    out_ref[...] = jnp.dot(x_ref[...], wt_ref[...],
                           preferred_element_type=jnp.float32) + bias_ref[...]


def _tc_project(x, wt, bias2d):
    grid = (_N // _ROWS,)
    return pl.pallas_call(
        _proj_body,
        grid=grid,
        in_specs=[
            pl.BlockSpec((_ROWS, _ENC), lambda i: (i, 0)),
            pl.BlockSpec((_ENC, _ENC), lambda i: (0, 0)),
            pl.BlockSpec((1, _ENC), lambda i: (0, 0)),
        ],
        out_specs=pl.BlockSpec((_ROWS, _ENC), lambda i: (i, 0)),
        out_shape=jax.ShapeDtypeStruct((_N, _ENC), jnp.float32),
    )(x, wt, bias2d)


def kernel(tokens, notes, genres, token_table, note_table, genre_table, W, b):
    gens_flat = jnp.repeat(genres, _T)
    (x,) = _sc_gather(token_table, tokens.reshape(-1), note_table,
                      notes.reshape(-1), genre_table, gens_flat)
    out = _tc_project(x, W.T, b.reshape(1, _ENC))
    return out.reshape(_B, _T, _ENC)


# speedup vs baseline: 2.2226x; 1.0892x over previous
import jax
import jax.numpy as jnp
from jax import lax
from jax.experimental import pallas as pl
from jax.experimental.pallas import tpu as pltpu
from jax.experimental.pallas import tpu_sc as plsc

_B = 4096
_T = 200
_N = _B * _T
_TOK_D = 64
_NOTE_D = 32
_GEN_D = 32
_ENC = 128

_NC = 2
_NS = 16
_NW = _NC * _NS
_TPW = _N // _NW
_CHUNK = 320
_NCH = _TPW // _CHUNK


def _gather_body(token_table, tokens, note_table, notes, genre_table, gens,
                 x_out,
                 tidx_a, nidx_a, gidx_a, trows_a, nrows_a, grows_a,
                 tidx_b, nidx_b, gidx_b, trows_b, nrows_b, grows_b,
                 sem_a, sem_b):
    wid = lax.axis_index("s") * _NC + lax.axis_index("c")
    base = wid * _TPW

    def start(chunk, tidx, nidx, gidx, trows, nrows, grows, sem):
        off = base + chunk * _CHUNK
        pltpu.sync_copy(tokens.at[pl.ds(off, _CHUNK)], tidx)
        pltpu.sync_copy(notes.at[pl.ds(off, _CHUNK)], nidx)
        pltpu.sync_copy(gens.at[pl.ds(off, _CHUNK)], gidx)
        pltpu.async_copy(token_table.at[tidx], trows, sem)
        pltpu.async_copy(note_table.at[nidx], nrows, sem)
        pltpu.async_copy(genre_table.at[gidx], grows, sem)

    def drain_and_write(chunk, tidx, nidx, gidx, trows, nrows, grows, sem):
        pltpu.make_async_copy(token_table.at[tidx], trows, sem).wait()
        pltpu.make_async_copy(note_table.at[nidx], nrows, sem).wait()
        pltpu.make_async_copy(genre_table.at[gidx], grows, sem).wait()
        off = base + chunk * _CHUNK
        pltpu.sync_copy(trows, x_out.at[pl.ds(off, _CHUNK), pl.ds(0, _TOK_D)])
        pltpu.sync_copy(nrows, x_out.at[pl.ds(off, _CHUNK),
                                        pl.ds(_TOK_D, _NOTE_D)])
        pltpu.sync_copy(grows, x_out.at[pl.ds(off, _CHUNK),
                                        pl.ds(_TOK_D + _NOTE_D, _GEN_D)])

    slot_a = (tidx_a, nidx_a, gidx_a, trows_a, nrows_a, grows_a, sem_a)
    slot_b = (tidx_b, nidx_b, gidx_b, trows_b, nrows_b, grows_b, sem_b)

    start(0, *slot_a)

    def step(j, carry):
        start(2 * j + 1, *slot_b)
        drain_and_write(2 * j, *slot_a)

        @pl.when(j < _NCH // 2 - 1)
        def _():
            start(2 * j + 2, *slot_a)

        drain_and_write(2 * j + 1, *slot_b)
        return carry

    lax.fori_loop(0, _NCH // 2, step, 0)


def _sc_gather(token_table, tokens_flat, note_table, notes_flat,
               genre_table, gens_flat):
    mesh = plsc.VectorSubcoreMesh(core_axis_name="c", subcore_axis_name="s")
    k = pl.kernel(
        _gather_body,
        mesh=mesh,
        compiler_params=pltpu.CompilerParams(use_tc_tiling_on_sc=False),
        out_type=[
            jax.ShapeDtypeStruct((_N, _ENC), jnp.float32),
        ],
        scratch_types=[
            pltpu.VMEM((_CHUNK,), jnp.int32),
            pltpu.VMEM((_CHUNK,), jnp.int32),
            pltpu.VMEM((_CHUNK,), jnp.int32),
            pltpu.VMEM((_CHUNK, _TOK_D), jnp.float32),
            pltpu.VMEM((_CHUNK, _NOTE_D), jnp.float32),
            pltpu.VMEM((_CHUNK, _GEN_D), jnp.float32),
            pltpu.VMEM((_CHUNK,), jnp.int32),
            pltpu.VMEM((_CHUNK,), jnp.int32),
            pltpu.VMEM((_CHUNK,), jnp.int32),
            pltpu.VMEM((_CHUNK, _TOK_D), jnp.float32),
            pltpu.VMEM((_CHUNK, _NOTE_D), jnp.float32),
            pltpu.VMEM((_CHUNK, _GEN_D), jnp.float32),
            pltpu.SemaphoreType.DMA,
            pltpu.SemaphoreType.DMA,
        ],
    )
    return k(token_table, tokens_flat, note_table, notes_flat,
             genre_table, gens_flat)


_ROWS = 1600


def _proj_body(x_ref, wt_ref, bias_ref, out_ref):
    out_ref[...] = jnp.dot(x_ref[...], wt_ref[...],
                           preferred_element_type=jnp.float32) + bias_ref[...]


def _tc_project(x, wt, bias2d):
    grid = (_N // _ROWS,)
    return pl.pallas_call(
        _proj_body,
        grid=grid,
        in_specs=[
            pl.BlockSpec((_ROWS, _ENC), lambda i: (i, 0)),
            pl.BlockSpec((_ENC, _ENC), lambda i: (0, 0)),
            pl.BlockSpec((1, _ENC), lambda i: (0, 0)),
        ],
        out_specs=pl.BlockSpec((_ROWS, _ENC), lambda i: (i, 0)),
        out_shape=jax.ShapeDtypeStruct((_N, _ENC), jnp.float32),
    )(x, wt, bias2d)


def kernel(tokens, notes, genres, token_table, note_table, genre_table, W, b):
    gens_flat = jnp.repeat(genres, _T)
    (x,) = _sc_gather(token_table, tokens.reshape(-1), note_table,
                      notes.reshape(-1), genre_table, gens_flat)
    out = _tc_project(x, W.T, b.reshape(1, _ENC))
    return out.reshape(_B, _T, _ENC)


# speedup vs baseline: 2.5070x; 1.1279x over previous
import jax
import jax.numpy as jnp
from jax import lax
from jax.experimental import pallas as pl
from jax.experimental.pallas import tpu as pltpu
from jax.experimental.pallas import tpu_sc as plsc

_B = 4096
_T = 200
_N = _B * _T
_TOK_D = 64
_NOTE_D = 32
_GEN_D = 32
_ENC = 128

_NC = 2
_NS = 16
_NW = _NC * _NS
_TPW = _N // _NW
_CHUNK = 320
_NCH = _TPW // _CHUNK


def _gather_body(token_table, tokens, note_table, notes, genre_table, gens,
                 x_out,
                 tidx_a, nidx_a, gidx_a, trows_a, nrows_a, grows_a,
                 tidx_b, nidx_b, gidx_b, trows_b, nrows_b, grows_b,
                 sem_a, sem_b):
    wid = lax.axis_index("s") * _NC + lax.axis_index("c")
    base = wid * _TPW

    def start(chunk, tidx, nidx, gidx, trows, nrows, grows, sem):
        off = base + chunk * _CHUNK
        pltpu.sync_copy(tokens.at[pl.ds(off, _CHUNK)], tidx)
        pltpu.sync_copy(notes.at[pl.ds(off, _CHUNK)], nidx)
        pltpu.sync_copy(gens.at[pl.ds(off, _CHUNK)], gidx)
        pltpu.async_copy(token_table.at[tidx], trows, sem)
        pltpu.async_copy(note_table.at[nidx], nrows, sem)
        pltpu.async_copy(genre_table.at[gidx], grows, sem)

    def drain_and_write(chunk, tidx, nidx, gidx, trows, nrows, grows, sem):
        pltpu.make_async_copy(token_table.at[tidx], trows, sem).wait()
        pltpu.make_async_copy(note_table.at[nidx], nrows, sem).wait()
        pltpu.make_async_copy(genre_table.at[gidx], grows, sem).wait()
        off = base + chunk * _CHUNK
        pltpu.sync_copy(trows, x_out.at[pl.ds(off, _CHUNK), pl.ds(0, _TOK_D)])
        pltpu.sync_copy(nrows, x_out.at[pl.ds(off, _CHUNK),
                                        pl.ds(_TOK_D, _NOTE_D)])
        pltpu.sync_copy(grows, x_out.at[pl.ds(off, _CHUNK),
                                        pl.ds(_TOK_D + _NOTE_D, _GEN_D)])

    slot_a = (tidx_a, nidx_a, gidx_a, trows_a, nrows_a, grows_a, sem_a)
    slot_b = (tidx_b, nidx_b, gidx_b, trows_b, nrows_b, grows_b, sem_b)

    start(0, *slot_a)

    def step(j, carry):
        start(2 * j + 1, *slot_b)
        drain_and_write(2 * j, *slot_a)

        @pl.when(j < _NCH // 2 - 1)
        def _():
            start(2 * j + 2, *slot_a)

        drain_and_write(2 * j + 1, *slot_b)
        return carry

    lax.fori_loop(0, _NCH // 2, step, 0)


def _sc_gather(token_table, tokens_flat, note_table, notes_flat,
               genre_table, gens_flat):
    mesh = plsc.VectorSubcoreMesh(core_axis_name="c", subcore_axis_name="s")
    k = pl.kernel(
        _gather_body,
        mesh=mesh,
        compiler_params=pltpu.CompilerParams(use_tc_tiling_on_sc=False),
        out_type=[
            jax.ShapeDtypeStruct((_N, _ENC), jnp.float32),
        ],
        scratch_types=[
            pltpu.VMEM((_CHUNK,), jnp.int32),
            pltpu.VMEM((_CHUNK,), jnp.int32),
            pltpu.VMEM((_CHUNK,), jnp.int32),
            pltpu.VMEM((_CHUNK, _TOK_D), jnp.float32),
            pltpu.VMEM((_CHUNK, _NOTE_D), jnp.float32),
            pltpu.VMEM((_CHUNK, _GEN_D), jnp.float32),
            pltpu.VMEM((_CHUNK,), jnp.int32),
            pltpu.VMEM((_CHUNK,), jnp.int32),
            pltpu.VMEM((_CHUNK,), jnp.int32),
            pltpu.VMEM((_CHUNK, _TOK_D), jnp.float32),
            pltpu.VMEM((_CHUNK, _NOTE_D), jnp.float32),
            pltpu.VMEM((_CHUNK, _GEN_D), jnp.float32),
            pltpu.SemaphoreType.DMA,
            pltpu.SemaphoreType.DMA,
        ],
    )
    return k(token_table, tokens_flat, note_table, notes_flat,
             genre_table, gens_flat)


_ROWS = 6400


def _proj_body(x_ref, wt_ref, bias_ref, out_ref):
    out_ref[...] = jnp.dot(x_ref[...], wt_ref[...],
                           preferred_element_type=jnp.float32) + bias_ref[...]


def _tc_project(x, wt, bias2d):
    grid = (_N // _ROWS,)
    return pl.pallas_call(
        _proj_body,
        grid=grid,
        in_specs=[
            pl.BlockSpec((_ROWS, _ENC), lambda i: (i, 0)),
            pl.BlockSpec((_ENC, _ENC), lambda i: (0, 0)),
            pl.BlockSpec((1, _ENC), lambda i: (0, 0)),
        ],
        out_specs=pl.BlockSpec((_ROWS, _ENC), lambda i: (i, 0)),
        out_shape=jax.ShapeDtypeStruct((_N, _ENC), jnp.float32),
    )(x, wt, bias2d)


def kernel(tokens, notes, genres, token_table, note_table, genre_table, W, b):
    gens_flat = jnp.repeat(genres, _T)
    (x,) = _sc_gather(token_table, tokens.reshape(-1), note_table,
                      notes.reshape(-1), genre_table, gens_flat)
    out = _tc_project(x, W.T, b.reshape(1, _ENC))
    return out.reshape(_B, _T, _ENC)


# speedup vs baseline: 2.5512x; 1.0176x over previous
import jax
import jax.numpy as jnp
from jax import lax
from jax.experimental import pallas as pl
from jax.experimental.pallas import tpu as pltpu
from jax.experimental.pallas import tpu_sc as plsc

_B = 4096
_T = 200
_N = _B * _T
_TOK_D = 64
_NOTE_D = 32
_GEN_D = 32
_ENC = 128

_NC = 2
_NS = 16
_NW = _NC * _NS
_CHUNK = 320

_S = 4
_NSL = _N // _S
_TPW = _NSL // _NW
_NCH = _TPW // _CHUNK


def _gather_body(token_table, tokens, note_table, notes, genre_table, gens,
                 x_out,
                 tidx_a, nidx_a, gidx_a, trows_a, nrows_a, grows_a,
                 tidx_b, nidx_b, gidx_b, trows_b, nrows_b, grows_b,
                 sem_a, sem_b):
    wid = lax.axis_index("s") * _NC + lax.axis_index("c")
    base = wid * _TPW

    def start(chunk, tidx, nidx, gidx, trows, nrows, grows, sem):
        off = base + chunk * _CHUNK
        pltpu.sync_copy(tokens.at[pl.ds(off, _CHUNK)], tidx)
        pltpu.sync_copy(notes.at[pl.ds(off, _CHUNK)], nidx)
        pltpu.sync_copy(gens.at[pl.ds(off, _CHUNK)], gidx)
        pltpu.async_copy(token_table.at[tidx], trows, sem)
        pltpu.async_copy(note_table.at[nidx], nrows, sem)
        pltpu.async_copy(genre_table.at[gidx], grows, sem)

    def drain_and_write(chunk, tidx, nidx, gidx, trows, nrows, grows, sem):
        pltpu.make_async_copy(token_table.at[tidx], trows, sem).wait()
        pltpu.make_async_copy(note_table.at[nidx], nrows, sem).wait()
        pltpu.make_async_copy(genre_table.at[gidx], grows, sem).wait()
        off = base + chunk * _CHUNK
        pltpu.sync_copy(trows, x_out.at[pl.ds(off, _CHUNK), pl.ds(0, _TOK_D)])
        pltpu.sync_copy(nrows, x_out.at[pl.ds(off, _CHUNK),
                                        pl.ds(_TOK_D, _NOTE_D)])
        pltpu.sync_copy(grows, x_out.at[pl.ds(off, _CHUNK),
                                        pl.ds(_TOK_D + _NOTE_D, _GEN_D)])

    slot_a = (tidx_a, nidx_a, gidx_a, trows_a, nrows_a, grows_a, sem_a)
    slot_b = (tidx_b, nidx_b, gidx_b, trows_b, nrows_b, grows_b, sem_b)

    start(0, *slot_a)

    def step(j, carry):
        start(2 * j + 1, *slot_b)
        drain_and_write(2 * j, *slot_a)

        @pl.when(j < _NCH // 2 - 1)
        def _():
            start(2 * j + 2, *slot_a)

        drain_and_write(2 * j + 1, *slot_b)
        return carry

    lax.fori_loop(0, _NCH // 2, step, 0)


def _sc_gather(token_table, tokens_sl, note_table, notes_sl,
               genre_table, gens_sl):
    mesh = plsc.VectorSubcoreMesh(core_axis_name="c", subcore_axis_name="s")
    k = pl.kernel(
        _gather_body,
        mesh=mesh,
        compiler_params=pltpu.CompilerParams(use_tc_tiling_on_sc=False),
        out_type=[
            jax.ShapeDtypeStruct((_NSL, _ENC), jnp.float32),
        ],
        scratch_types=[
            pltpu.VMEM((_CHUNK,), jnp.int32),
            pltpu.VMEM((_CHUNK,), jnp.int32),
            pltpu.VMEM((_CHUNK,), jnp.int32),
            pltpu.VMEM((_CHUNK, _TOK_D), jnp.float32),
            pltpu.VMEM((_CHUNK, _NOTE_D), jnp.float32),
            pltpu.VMEM((_CHUNK, _GEN_D), jnp.float32),
            pltpu.VMEM((_CHUNK,), jnp.int32),
            pltpu.VMEM((_CHUNK,), jnp.int32),
            pltpu.VMEM((_CHUNK,), jnp.int32),
            pltpu.VMEM((_CHUNK, _TOK_D), jnp.float32),
            pltpu.VMEM((_CHUNK, _NOTE_D), jnp.float32),
            pltpu.VMEM((_CHUNK, _GEN_D), jnp.float32),
            pltpu.SemaphoreType.DMA,
            pltpu.SemaphoreType.DMA,
        ],
    )
    return k(token_table, tokens_sl, note_table, notes_sl,
             genre_table, gens_sl)


_ROWS = 6400
_BPS = _NSL // _ROWS


def _proj_body(x_ref, wt_ref, bias_ref, prev_ref, out_ref):
    del prev_ref
    out_ref[...] = jnp.dot(x_ref[...], wt_ref[...],
                           preferred_element_type=jnp.float32) + bias_ref[...]


def _tc_project_slice(x_sl, wt, bias2d, out_prev, s):
    return pl.pallas_call(
        _proj_body,
        grid=(_BPS,),
        in_specs=[
            pl.BlockSpec((_ROWS, _ENC), lambda i: (i, 0)),
            pl.BlockSpec((_ENC, _ENC), lambda i: (0, 0)),
            pl.BlockSpec((1, _ENC), lambda i: (0, 0)),
            pl.BlockSpec(memory_space=pl.ANY),
        ],
        out_specs=pl.BlockSpec((_ROWS, _ENC),
                               lambda i, s=s: (s * _BPS + i, 0)),
        out_shape=jax.ShapeDtypeStruct((_N, _ENC), jnp.float32),
        input_output_aliases={3: 0},
    )(x_sl, wt, bias2d, out_prev)


def _proj_body_first(x_ref, wt_ref, bias_ref, out_ref):
    out_ref[...] = jnp.dot(x_ref[...], wt_ref[...],
                           preferred_element_type=jnp.float32) + bias_ref[...]


def kernel(tokens, notes, genres, token_table, note_table, genre_table, W, b):
    gens_flat = jnp.repeat(genres, _T)
    tokens_flat = tokens.reshape(-1)
    notes_flat = notes.reshape(-1)
    wt = W.T
    bias2d = b.reshape(1, _ENC)

    xs = []
    for s in range(_S):
        (x_s,) = _sc_gather(token_table, lax.dynamic_slice(tokens_flat,
                                                           (s * _NSL,),
                                                           (_NSL,)),
                            note_table, lax.dynamic_slice(notes_flat,
                                                          (s * _NSL,),
                                                          (_NSL,)),
                            genre_table, lax.dynamic_slice(gens_flat,
                                                           (s * _NSL,),
                                                           (_NSL,)))
        xs.append(x_s)

    out = None
    for s in range(_S):
        if out is None:
            out = pl.pallas_call(
                _proj_body_first,
                grid=(_BPS,),
                in_specs=[
                    pl.BlockSpec((_ROWS, _ENC), lambda i: (i, 0)),
                    pl.BlockSpec((_ENC, _ENC), lambda i: (0, 0)),
                    pl.BlockSpec((1, _ENC), lambda i: (0, 0)),
                ],
                out_specs=pl.BlockSpec((_ROWS, _ENC), lambda i: (i, 0)),
                out_shape=jax.ShapeDtypeStruct((_N, _ENC), jnp.float32),
            )(xs[s], wt, bias2d)
        else:
            out = _tc_project_slice(xs[s], wt, bias2d, out, s)
    return out.reshape(_B, _T, _ENC)
